# baseline (speedup 1.0000x reference)
import functools

import jax
import jax.numpy as jnp
from jax import lax
from jax.experimental import pallas as pl
from jax.experimental.pallas import tpu as pltpu
from jax.experimental.pallas import tpu_sc as plsc

ROWS = 4096
COLS = 512
TOTAL = ROWS * COLS
K = TOTAL // 4
NBINS = 32768
SHIFT = 16
HR, HC = 256, 128
NW = 32
ROWS_PER_W = ROWS // NW
CHUNK_ROWS = 16
NCHUNK = ROWS_PER_W // CHUNK_ROWS
VECS = CHUNK_ROWS * COLS // 16
GRID_A = 4


def _bce_block(p_ref, t_ref, o_ref):
    x = p_ref[...]
    t = t_ref[...]
    o_ref[...] = jnp.maximum(x, 0.0) - x * t + jnp.log1p(jnp.exp(-jnp.abs(x)))


def _bce(pred2d, target2d):
    return pl.pallas_call(
        _bce_block,
        grid=(GRID_A,),
        in_specs=[
            pl.BlockSpec((ROWS // GRID_A, COLS), lambda i: (i, 0)),
            pl.BlockSpec((ROWS // GRID_A, COLS), lambda i: (i, 0)),
        ],
        out_specs=pl.BlockSpec((ROWS // GRID_A, COLS), lambda i: (i, 0)),
        out_shape=jax.ShapeDtypeStruct((ROWS, COLS), jnp.float32),
    )(pred2d, target2d)


def _sc_hist_body(loss_hbm, out_hbm, buf0, buf1, hist, sem0, sem1):
    wid = lax.axis_index("s") * 2 + lax.axis_index("c")
    row0 = wid * ROWS_PER_W
    bufs = (buf0, buf1)
    sems = (sem0, sem1)

    def copy(c):
        return pltpu.make_async_copy(
            loss_hbm.at[pl.ds(row0 + c * CHUNK_ROWS, CHUNK_ROWS), :],
            bufs[c % 2], sems[c % 2])

    copy(0).start()
    copy(1).start()

    zeros16 = jnp.zeros((16,), jnp.int32)

    def zinit(i, c):
        for u in range(8):
            hist[i, pl.ds(u * 16, 16)] = zeros16
        return c

    lax.fori_loop(0, HR, zinit, 0)

    ones16 = jnp.ones((16,), jnp.int32)
    U = 16
    for c in range(NCHUNK):
        copy(c).wait()
        if c + 2 < NCHUNK:
            copy(c + 2).start()
        buf = bufs[c % 2]

        def vec_body(i, cc, buf=buf):
            row = (i * U) // 32
            cb = ((i * U) % 32) * 16
            xs = [buf[row, pl.ds(cb + u * 16, 16)] for u in range(U)]
            bins = [
                lax.shift_right_logical(
                    lax.bitcast_convert_type(x, jnp.int32), SHIFT)
                for x in xs
            ]
            for b in bins:
                r = lax.shift_right_logical(b, 7)
                col = lax.bitwise_and(b, 127)
                plsc.addupdate_scatter(hist, [r, col], ones16)
            return cc

        lax.fori_loop(0, VECS // U, vec_body, 0)
    pltpu.sync_copy(hist, out_hbm.at[wid])


def _sc_hist(loss2d):
    mesh = plsc.VectorSubcoreMesh(core_axis_name="c", subcore_axis_name="s")
    kfn = functools.partial(
        pl.kernel,
        mesh=mesh,
        out_type=jax.ShapeDtypeStruct((NW, HR, HC), jnp.int32),
        scratch_types=[
            pltpu.VMEM((CHUNK_ROWS, COLS), jnp.float32),
            pltpu.VMEM((CHUNK_ROWS, COLS), jnp.float32),
            pltpu.VMEM((HR, HC), jnp.int32),
            pltpu.SemaphoreType.DMA,
            pltpu.SemaphoreType.DMA,
        ],
        compiler_params=pltpu.CompilerParams(needs_layout_passes=False),
    )(_sc_hist_body)
    return kfn(loss2d)


GRID_C = 8


def _select_body(hist_hbm, loss_ref, o_ref, hist_vmem, th_ref, acc_ref, sem):
    i = pl.program_id(0)
    kf = jnp.float32(K)

    @pl.when(i == 0)
    def _search():
        pltpu.make_async_copy(hist_hbm, hist_vmem, sem).start()
        pltpu.make_async_copy(hist_hbm, hist_vmem, sem).wait()
        hs = hist_vmem[...].astype(jnp.float32)
        hbins = jnp.sum(hs, axis=0)
        rows = lax.broadcasted_iota(jnp.int32, (HR, HC), 0)
        cols = lax.broadcasted_iota(jnp.int32, (HR, HC), 1)
        lin = rows * HC + cols

        def search(_, lohi):
            lo, hi = lohi
            mid = (lo + hi) // 2
            c = jnp.sum(jnp.where(lin >= mid, hbins, 0.0))
            ok = c >= kf
            return jnp.where(ok, mid, lo), jnp.where(ok, hi, mid)

        lo, _ = lax.fori_loop(0, 15, search, (jnp.int32(0), jnp.int32(NBINS)))
        t_lo = lax.bitcast_convert_type(lo << SHIFT, jnp.float32)
        t_hi = lax.bitcast_convert_type((lo + 1) << SHIFT, jnp.float32)
        th_ref[0] = t_hi
        th_ref[1] = 0.5 * (t_lo + t_hi)
        acc_ref[0] = 0.0
        acc_ref[1] = 0.0

    t_hi = th_ref[0]
    x = loss_ref[...]
    m = x >= t_hi
    acc_ref[0] += jnp.sum(jnp.where(m, x, 0.0))
    acc_ref[1] += jnp.sum(jnp.where(m, 1.0, 0.0))

    @pl.when(i == GRID_C - 1)
    def _emit():
        s_ge = acc_ref[0]
        n_ge = acc_ref[1]
        fill = th_ref[1]
        o_ref[...] = jnp.reshape((s_ge + (kf - n_ge) * fill) / kf, (1, 1))


def _select(hists, loss2d):
    return pl.pallas_call(
        _select_body,
        grid=(GRID_C,),
        in_specs=[
            pl.BlockSpec(memory_space=pltpu.MemorySpace.HBM),
            pl.BlockSpec((ROWS // GRID_C, COLS), lambda i: (i, 0)),
        ],
        out_specs=pl.BlockSpec((1, 1), lambda i: (0, 0)),
        out_shape=jax.ShapeDtypeStruct((1, 1), jnp.float32),
        scratch_shapes=[
            pltpu.VMEM((NW, HR, HC), jnp.int32),
            pltpu.SMEM((2,), jnp.float32),
            pltpu.SMEM((2,), jnp.float32),
            pltpu.SemaphoreType.DMA,
        ],
    )(hists, loss2d)


def kernel(pred, target):
    p2 = pred.reshape(ROWS, COLS)
    t2 = target.reshape(ROWS, COLS)
    loss = _bce(p2, t2)
    hists = _sc_hist(loss)
    out = _select(hists, loss)
    return out[0, 0]


# speedup vs baseline: 1.0555x; 1.0555x over previous
import functools

import jax
import jax.numpy as jnp
from jax import lax
from jax.experimental import pallas as pl
from jax.experimental.pallas import tpu as pltpu
from jax.experimental.pallas import tpu_sc as plsc

ROWS = 4096
COLS = 512
TOTAL = ROWS * COLS
K = TOTAL // 4
NBINS = 32768
SHIFT = 16
HR, HC = 256, 128
NW = 32
ROWS_PER_W = ROWS // NW
CHUNK_ROWS = 16
NCHUNK = ROWS_PER_W // CHUNK_ROWS
VECS = CHUNK_ROWS * COLS // 16
GRID_A = 4


def _bce_block(p_ref, t_ref, o_ref):
    x = p_ref[...]
    t = t_ref[...]
    o_ref[...] = jnp.maximum(x, 0.0) - x * t + jnp.log1p(jnp.exp(-jnp.abs(x)))


def _bce(pred2d, target2d):
    return pl.pallas_call(
        _bce_block,
        grid=(GRID_A,),
        in_specs=[
            pl.BlockSpec((ROWS // GRID_A, COLS), lambda i: (i, 0)),
            pl.BlockSpec((ROWS // GRID_A, COLS), lambda i: (i, 0)),
        ],
        out_specs=pl.BlockSpec((ROWS // GRID_A, COLS), lambda i: (i, 0)),
        out_shape=jax.ShapeDtypeStruct((ROWS, COLS), jnp.float32),
    )(pred2d, target2d)


def _sc_hist_body(loss_hbm, out_hbm, buf0, buf1, hist, sem0, sem1):
    wid = lax.axis_index("s") * 2 + lax.axis_index("c")
    row0 = wid * ROWS_PER_W
    bufs = (buf0, buf1)
    sems = (sem0, sem1)

    def copy(c):
        return pltpu.make_async_copy(
            loss_hbm.at[pl.ds(row0 + c * CHUNK_ROWS, CHUNK_ROWS), :],
            bufs[c % 2], sems[c % 2])

    copy(0).start()
    copy(1).start()

    zeros16 = jnp.zeros((16,), jnp.int32)

    def zinit(i, c):
        for u in range(8):
            hist[i, pl.ds(u * 16, 16)] = zeros16
        return c

    lax.fori_loop(0, HR, zinit, 0)

    ones16 = jnp.ones((16,), jnp.int32)
    U = 16
    for c in range(NCHUNK):
        copy(c).wait()
        if c + 2 < NCHUNK:
            copy(c + 2).start()
        buf = bufs[c % 2]

        def vec_body(i, cc, buf=buf):
            row = (i * U) // 32
            cb = ((i * U) % 32) * 16
            xs = [buf[row, pl.ds(cb + u * 16, 16)] for u in range(U)]
            bins = [
                lax.shift_right_logical(
                    lax.bitcast_convert_type(x, jnp.int32), SHIFT)
                for x in xs
            ]
            for b in bins:
                r = lax.shift_right_logical(b, 7)
                col = lax.bitwise_and(b, 127)
                plsc.addupdate_scatter(hist, [r, col], ones16)
            return cc

        lax.fori_loop(0, VECS // U, vec_body, 0)
    pltpu.sync_copy(hist, out_hbm.at[wid])


def _sc_hist(loss2d):
    mesh = plsc.VectorSubcoreMesh(core_axis_name="c", subcore_axis_name="s")
    kfn = functools.partial(
        pl.kernel,
        mesh=mesh,
        out_type=jax.ShapeDtypeStruct((NW, HR, HC), jnp.int32),
        scratch_types=[
            pltpu.VMEM((CHUNK_ROWS, COLS), jnp.float32),
            pltpu.VMEM((CHUNK_ROWS, COLS), jnp.float32),
            pltpu.VMEM((HR, HC), jnp.int32),
            pltpu.SemaphoreType.DMA,
            pltpu.SemaphoreType.DMA,
        ],
        compiler_params=pltpu.CompilerParams(needs_layout_passes=False),
    )(_sc_hist_body)
    return kfn(loss2d)


def _select_body(hist_ref, loss_ref, o_ref):
    hs = hist_ref[...].astype(jnp.float32)
    hbins = jnp.sum(hs, axis=0)
    rows = lax.broadcasted_iota(jnp.int32, (HR, HC), 0)
    cols = lax.broadcasted_iota(jnp.int32, (HR, HC), 1)
    lin = rows * HC + cols
    kf = jnp.float32(K)

    def search(_, lohi):
        lo, hi = lohi
        mid = (lo + hi) // 2
        c = jnp.sum(jnp.where(lin >= mid, hbins, 0.0))
        ok = c >= kf
        return jnp.where(ok, mid, lo), jnp.where(ok, hi, mid)

    lo, _ = lax.fori_loop(0, 15, search, (jnp.int32(0), jnp.int32(NBINS)))
    t_lo = lax.bitcast_convert_type(lo << SHIFT, jnp.float32)
    t_hi = lax.bitcast_convert_type((lo + 1) << SHIFT, jnp.float32)

    x = loss_ref[...]
    m = x >= t_hi
    n_ge = jnp.sum(jnp.where(m, 1.0, 0.0))
    s_ge = jnp.sum(jnp.where(m, x, 0.0))
    fill = 0.5 * (t_lo + t_hi)
    o_ref[...] = jnp.reshape((s_ge + (kf - n_ge) * fill) / kf, (1, 1))


def _select(hists, loss2d):
    return pl.pallas_call(
        _select_body,
        in_specs=[
            pl.BlockSpec((NW, HR, HC), lambda: (0, 0, 0)),
            pl.BlockSpec((ROWS, COLS), lambda: (0, 0)),
        ],
        out_specs=pl.BlockSpec((1, 1), lambda: (0, 0)),
        out_shape=jax.ShapeDtypeStruct((1, 1), jnp.float32),
    )(hists, loss2d)


def kernel(pred, target):
    p2 = pred.reshape(ROWS, COLS)
    t2 = target.reshape(ROWS, COLS)
    loss = _bce(p2, t2)
    hists = _sc_hist(loss)
    out = _select(hists, loss)
    return out[0, 0]
